# baseline (speedup 1.0000x reference)
import functools

import jax
import jax.numpy as jnp
from jax import lax
from jax.experimental import pallas as pl
from jax.experimental.pallas import tpu as pltpu
from jax.experimental.pallas import tpu_sc as plsc

EMBD = 64
NC, NS = 2, 16
NW = NC * NS

_B = 4096 * 200
_BPW = _B // NW
_C = 512
_NCHUNK = _BPW // _C

_mesh = plsc.VectorSubcoreMesh(
    core_axis_name="c", subcore_axis_name="s", num_cores=NC, num_subcores=NS
)


@functools.partial(
    pl.kernel,
    out_type=jax.ShapeDtypeStruct((_B, EMBD), jnp.float32),
    mesh=_mesh,
    compiler_params=pltpu.CompilerParams(use_tc_tiling_on_sc=False),
    scratch_types=[
        pltpu.VMEM((_BPW,), jnp.int32),
        pltpu.VMEM((_C, EMBD), jnp.float32),
        pltpu.VMEM((_C, EMBD), jnp.float32),
        pltpu.SemaphoreType.DMA,
        pltpu.SemaphoreType.DMA,
    ],
)
def _embed_sc(idx_hbm, table_hbm, out_hbm, idx_v, rows0, rows1, sem0, sem1):
    wid = lax.axis_index("s") * NC + lax.axis_index("c")
    base = wid * _BPW
    pltpu.sync_copy(idx_hbm.at[pl.ds(base, _BPW)], idx_v)

    rows = (rows0, rows1)
    sems = (sem0, sem1)

    def start(g, b):
        pltpu.async_copy(table_hbm.at[idx_v.at[pl.ds(g * _C, _C)]], rows[b], sems[b])

    def wait(b):
        pltpu.make_async_copy(table_hbm.at[pl.ds(0, _C)], rows[b], sems[b]).wait()

    def put(g, b):
        pltpu.sync_copy(rows[b], out_hbm.at[pl.ds(base + g * _C, _C)])

    start(0, 0)

    @pl.loop(0, _NCHUNK, step=2)
    def _(g):
        start(g + 1, 1)
        wait(0)
        put(g, 0)

        @pl.when(g + 2 < _NCHUNK)
        def _():
            start(g + 2, 0)

        wait(1)
        put(g + 1, 1)


def kernel(x, table):
    idx = x.reshape(-1).astype(jnp.int32)
    out = _embed_sc(idx, table)
    return out.reshape(x.shape + (EMBD,))
